# baseline (speedup 1.0000x reference)
import functools

import jax
import jax.numpy as jnp
from jax import lax
from jax.experimental import pallas as pl
from jax.experimental.pallas import tpu as pltpu
from jax.experimental.pallas import tpu_sc as plsc

_NUM_CLASSES = 21
_N = 1048576
_NW = 32
_CHUNK = 2048
_NCHUNKS = 12
_N_SC = _NW * _CHUNK * _NCHUNKS
_GROUPS = _CHUNK // 16
_HIST_PAD = 448
_BT = 16384
_TC_BLOCKS = (_N - _N_SC) // _BT
_TC_OFF = _N_SC // _BT


def _argmax16(chunk_ref, r0):
    nodes = []
    for c in range(_NUM_CLASSES):
        v = chunk_ref[c, pl.ds(r0, 16)]
        nodes.append((v, jnp.full((16,), c, jnp.int32)))
    while len(nodes) > 1:
        nxt = []
        for i in range(0, len(nodes) - 1, 2):
            (va, ia), (vb, ib) = nodes[i], nodes[i + 1]
            m = vb > va
            nxt.append((jnp.where(m, vb, va), jnp.where(m, ib, ia)))
        if len(nodes) % 2:
            nxt.append(nodes[-1])
        nodes = nxt
    return nodes[0][1]


def _cm_body(out_hbm, tgt_hbm, part_hbm, chunk0, chunk1, tgt0, tgt1,
             hist_v, res_v, sem0, sem1):
    s_id = lax.axis_index("s")
    wid = lax.axis_index("c") * 16 + s_id
    lanes = lax.broadcasted_iota(jnp.int32, (16,), 0)
    zeros_f = jnp.zeros((16,), jnp.float32)
    ones_f = jnp.ones((16,), jnp.float32)
    chunks = (chunk0, chunk1)
    tgts = (tgt0, tgt1)
    sems = (sem0, sem1)

    def zero_body(i, carry):
        hist_v[pl.ds(i * 16, 16)] = zeros_f
        return carry

    lax.fori_loop(0, (16 * _HIST_PAD) // 16, zero_body, 0)

    def start(ci, buf):
        base = (wid * _NCHUNKS + ci) * _CHUNK
        a = pltpu.async_copy(
            out_hbm.at[:, pl.ds(base, _CHUNK)], chunks[buf], sems[buf])
        b = pltpu.async_copy(
            tgt_hbm.at[pl.ds(base, _CHUNK)], tgts[buf], sems[buf])
        return a, b

    pending = start(0, 0)
    for ci in range(_NCHUNKS):
        buf = ci % 2
        for d in pending:
            d.wait()
        if ci + 1 < _NCHUNKS:
            pending = start(ci + 1, buf ^ 1)
        chunk_v, tgt_v = chunks[buf], tgts[buf]

        def group_body(g, inner):
            r0 = g * 16
            arg = _argmax16(chunk_v, r0)
            t = tgt_v[pl.ds(r0, 16)]
            flat = lanes * _HIST_PAD + t * _NUM_CLASSES + arg
            plsc.addupdate_scatter(hist_v, [flat], ones_f)
            return inner

        lax.fori_loop(0, _GROUPS, group_body, 0)

    def red_body(b, carry):
        acc = zeros_f
        for l in range(16):
            acc = acc + hist_v[pl.ds(l * _HIST_PAD + b * 16, 16)]
        res_v[pl.ds(b * 16, 16)] = acc
        return carry

    lax.fori_loop(0, _HIST_PAD // 16, red_body, 0)
    pltpu.sync_copy(res_v, part_hbm.at[pl.ds(wid * _HIST_PAD, _HIST_PAD)])


def _tc_body(x_ref, t_ref, o_ref):
    x = x_ref[...]
    cls = lax.broadcasted_iota(jnp.int32, (_NUM_CLASSES, _BT), 0)
    maxv = jnp.max(x, axis=0)
    am = jnp.min(jnp.where(x == maxv[None, :], cls, _NUM_CLASSES), axis=0)
    ohp = (cls == am[None, :]).astype(jnp.float32)
    oht = (cls == t_ref[...]).astype(jnp.float32)
    cm = lax.dot_general(oht, ohp, (((1,), (1,)), ((), ())),
                         preferred_element_type=jnp.float32)

    @pl.when(pl.program_id(0) == 0)
    def _():
        o_ref[...] = jnp.zeros_like(o_ref)

    o_ref[...] += cm


@jax.jit
def kernel(output, target):
    out_t = output.T
    mesh = plsc.VectorSubcoreMesh(core_axis_name="c", subcore_axis_name="s")
    run = functools.partial(
        pl.kernel,
        mesh=mesh,
        out_type=jax.ShapeDtypeStruct((_NW * _HIST_PAD,), jnp.float32),
        scratch_types=[
            pltpu.VMEM((_NUM_CLASSES, _CHUNK), jnp.float32),
            pltpu.VMEM((_NUM_CLASSES, _CHUNK), jnp.float32),
            pltpu.VMEM((_CHUNK,), jnp.int32),
            pltpu.VMEM((_CHUNK,), jnp.int32),
            pltpu.VMEM((16 * _HIST_PAD,), jnp.float32),
            pltpu.VMEM((_HIST_PAD,), jnp.float32),
            pltpu.SemaphoreType.DMA,
            pltpu.SemaphoreType.DMA,
        ],
        compiler_params=pltpu.CompilerParams(
            needs_layout_passes=False, use_tc_tiling_on_sc=True),
    )(_cm_body)
    parts = run(out_t, target)

    tc_cm = pl.pallas_call(
        _tc_body,
        grid=(_TC_BLOCKS,),
        in_specs=[
            pl.BlockSpec((_NUM_CLASSES, _BT), lambda i: (0, _TC_OFF + i)),
            pl.BlockSpec((1, _BT), lambda i: (0, _TC_OFF + i)),
        ],
        out_specs=pl.BlockSpec((_NUM_CLASSES, _NUM_CLASSES), lambda i: (0, 0)),
        out_shape=jax.ShapeDtypeStruct((_NUM_CLASSES, _NUM_CLASSES),
                                       jnp.float32),
    )(out_t, target.reshape(1, _N))

    sc_cm = parts.reshape(_NW, _HIST_PAD).sum(axis=0)[
        : _NUM_CLASSES * _NUM_CLASSES].reshape(_NUM_CLASSES, _NUM_CLASSES)
    return sc_cm + tc_cm


# speedup vs baseline: 1.0894x; 1.0894x over previous
import functools

import jax
import jax.numpy as jnp
from jax import lax
from jax.experimental import pallas as pl
from jax.experimental.pallas import tpu as pltpu
from jax.experimental.pallas import tpu_sc as plsc

_NUM_CLASSES = 21
_N = 1048576
_NW = 32
_CHUNK = 2048
_NCHUNKS = 10
_N_SC = _NW * _CHUNK * _NCHUNKS
_GROUPS = _CHUNK // 16
_HIST_PAD = 448
_BT = 16384
_TC_BLOCKS = (_N - _N_SC) // _BT
_TC_OFF = _N_SC // _BT


def _argmax16(chunk_ref, r0):
    nodes = []
    for c in range(_NUM_CLASSES):
        v = chunk_ref[c, pl.ds(r0, 16)]
        nodes.append((v, jnp.full((16,), c, jnp.int32)))
    while len(nodes) > 1:
        nxt = []
        for i in range(0, len(nodes) - 1, 2):
            (va, ia), (vb, ib) = nodes[i], nodes[i + 1]
            m = vb > va
            nxt.append((jnp.where(m, vb, va), jnp.where(m, ib, ia)))
        if len(nodes) % 2:
            nxt.append(nodes[-1])
        nodes = nxt
    return nodes[0][1]


def _cm_body(out_hbm, tgt_hbm, part_hbm, chunk0, chunk1, tgt0, tgt1,
             hist_v, res_v, sem0, sem1):
    s_id = lax.axis_index("s")
    wid = lax.axis_index("c") * 16 + s_id
    lanes = lax.broadcasted_iota(jnp.int32, (16,), 0)
    zeros_f = jnp.zeros((16,), jnp.float32)
    ones_f = jnp.ones((16,), jnp.float32)
    chunks = (chunk0, chunk1)
    tgts = (tgt0, tgt1)
    sems = (sem0, sem1)

    def zero_body(i, carry):
        hist_v[pl.ds(i * 16, 16)] = zeros_f
        return carry

    lax.fori_loop(0, (16 * _HIST_PAD) // 16, zero_body, 0)

    def start(ci, buf):
        base = (wid * _NCHUNKS + ci) * _CHUNK
        a = pltpu.async_copy(
            out_hbm.at[:, pl.ds(base, _CHUNK)], chunks[buf], sems[buf])
        b = pltpu.async_copy(
            tgt_hbm.at[pl.ds(base, _CHUNK)], tgts[buf], sems[buf])
        return a, b

    pending = start(0, 0)
    for ci in range(_NCHUNKS):
        buf = ci % 2
        for d in pending:
            d.wait()
        if ci + 1 < _NCHUNKS:
            pending = start(ci + 1, buf ^ 1)
        chunk_v, tgt_v = chunks[buf], tgts[buf]

        def group_body(g, inner):
            r0 = g * 16
            arg = _argmax16(chunk_v, r0)
            t = tgt_v[pl.ds(r0, 16)]
            flat = lanes * _HIST_PAD + t * _NUM_CLASSES + arg
            plsc.addupdate_scatter(hist_v, [flat], ones_f)
            return inner

        lax.fori_loop(0, _GROUPS, group_body, 0)

    def red_body(b, carry):
        acc = zeros_f
        for l in range(16):
            acc = acc + hist_v[pl.ds(l * _HIST_PAD + b * 16, 16)]
        res_v[pl.ds(b * 16, 16)] = acc
        return carry

    lax.fori_loop(0, _HIST_PAD // 16, red_body, 0)
    pltpu.sync_copy(res_v, part_hbm.at[pl.ds(wid * _HIST_PAD, _HIST_PAD)])


def _tc_body(x_ref, t_ref, o_ref):
    x = x_ref[...]
    cls = lax.broadcasted_iota(jnp.int32, (_NUM_CLASSES, _BT), 0)
    maxv = jnp.max(x, axis=0)
    am = jnp.min(jnp.where(x == maxv[None, :], cls, _NUM_CLASSES), axis=0)
    ohp = (cls == am[None, :]).astype(jnp.float32)
    oht = (cls == t_ref[...]).astype(jnp.float32)
    cm = lax.dot_general(oht, ohp, (((1,), (1,)), ((), ())),
                         preferred_element_type=jnp.float32)

    @pl.when(pl.program_id(0) == 0)
    def _():
        o_ref[...] = jnp.zeros_like(o_ref)

    o_ref[...] += cm


@jax.jit
def kernel(output, target):
    out_t = output.T
    mesh = plsc.VectorSubcoreMesh(core_axis_name="c", subcore_axis_name="s")
    run = functools.partial(
        pl.kernel,
        mesh=mesh,
        out_type=jax.ShapeDtypeStruct((_NW * _HIST_PAD,), jnp.float32),
        scratch_types=[
            pltpu.VMEM((_NUM_CLASSES, _CHUNK), jnp.float32),
            pltpu.VMEM((_NUM_CLASSES, _CHUNK), jnp.float32),
            pltpu.VMEM((_CHUNK,), jnp.int32),
            pltpu.VMEM((_CHUNK,), jnp.int32),
            pltpu.VMEM((16 * _HIST_PAD,), jnp.float32),
            pltpu.VMEM((_HIST_PAD,), jnp.float32),
            pltpu.SemaphoreType.DMA,
            pltpu.SemaphoreType.DMA,
        ],
        compiler_params=pltpu.CompilerParams(
            needs_layout_passes=False, use_tc_tiling_on_sc=True),
    )(_cm_body)
    parts = run(out_t, target)

    tc_cm = pl.pallas_call(
        _tc_body,
        grid=(_TC_BLOCKS,),
        in_specs=[
            pl.BlockSpec((_NUM_CLASSES, _BT), lambda i: (0, _TC_OFF + i)),
            pl.BlockSpec((1, _BT), lambda i: (0, _TC_OFF + i)),
        ],
        out_specs=pl.BlockSpec((_NUM_CLASSES, _NUM_CLASSES), lambda i: (0, 0)),
        out_shape=jax.ShapeDtypeStruct((_NUM_CLASSES, _NUM_CLASSES),
                                       jnp.float32),
    )(out_t, target.reshape(1, _N))

    sc_cm = parts.reshape(_NW, _HIST_PAD).sum(axis=0)[
        : _NUM_CLASSES * _NUM_CLASSES].reshape(_NUM_CLASSES, _NUM_CLASSES)
    return sc_cm + tc_cm
